# baseline (speedup 1.0000x reference)
import functools

import jax
import jax.numpy as jnp
from jax import lax
from jax.experimental import pallas as pl
from jax.experimental.pallas import tpu as pltpu
from jax.experimental.pallas import tpu_sc as plsc

B = 16384
L = 200
D = 32
NC, NS, LANES = 2, 16, 16
NW = NC * NS
BPW = B // NW
SGR = 64
NSG = BPW // SGR
LH = L // 2
LHA = 96
LHB = 104
NBUF = 8

_mesh = plsc.VectorSubcoreMesh(
    core_axis_name="c", subcore_axis_name="s", num_cores=NC, num_subcores=NS
)


@functools.partial(
    pl.kernel,
    out_type=jax.ShapeDtypeStruct((B * D,), jnp.float32),
    mesh=_mesh,
    compiler_params=pltpu.CompilerParams(use_tc_tiling_on_sc=False),
    scratch_types=[
        pltpu.VMEM((2, 2 * SGR * LH), jnp.int32),
        pltpu.VMEM((BPW,), jnp.int32),
        pltpu.VMEM((BPW,), jnp.float32),
        pltpu.VMEM((NBUF, L, D), jnp.float32),
        pltpu.VMEM((2, SGR * D), jnp.float32),
        [pltpu.SemaphoreType.DMA] * NBUF,
        pltpu.SemaphoreType.DMA,
        [pltpu.SemaphoreType.DMA] * 2,
    ],
)
def _pool_kernel(ids_hbm, len_hbm, table_hbm, out_hbm,
                 ids_v, len_v, rcp_v, rows_v, out_v, sems, sem_ids, semo):
    wid = lax.axis_index("s") * NC + lax.axis_index("c")
    base = wid * BPW

    pltpu.sync_copy(len_hbm.at[pl.ds(base, BPW)], len_v)

    def rcp_body(i, carry):
        lenf = len_v[pl.ds(i * LANES, LANES)].astype(jnp.float32)
        rcp_v[pl.ds(i * LANES, LANES)] = 1.0 / jnp.maximum(lenf, 1.0)
        return carry

    lax.fori_loop(0, BPW // LANES, rcp_body, 0)

    def ids_load_desc(s):
        return pltpu.make_async_copy(
            ids_hbm.at[pl.ds((base + s * SGR) * L, SGR * L)],
            ids_v.at[lax.rem(s, 2)], sem_ids)

    def out_write_desc(s, p):
        return pltpu.make_async_copy(
            out_v.at[p], out_hbm.at[pl.ds((base + s * SGR) * D, SGR * D)], semo[p])

    def out_write_op(s, op):
        for p in range(2):
            @pl.when(lax.rem(s, 2) == p)
            def _():
                op(out_write_desc(s, p))

    def gather_descs(r, buf):
        p = lax.rem(r // SGR, 2)
        ro = lax.rem(r, SGR)
        return (
            pltpu.make_async_copy(
                table_hbm.at[ids_v.at[p, pl.ds(ro * L, LHA)]],
                rows_v.at[buf, pl.ds(0, LHA)], sems[buf]),
            pltpu.make_async_copy(
                table_hbm.at[ids_v.at[p, pl.ds(ro * L + LHA, LHB)]],
                rows_v.at[buf, pl.ds(LHA, LHB)], sems[buf]),
        )

    def issue(r, buf):
        for h in gather_descs(r, buf):
            h.start()

    def wait(r, buf):
        for h in gather_descs(r, buf):
            h.wait()

    def process(r, buf):
        U = 8
        z = jnp.zeros((LANES,), jnp.float32)

        def j_body(j, accs):
            accs = list(accs)
            jb = j * U
            for t in range(U):
                accs[2 * t] = accs[2 * t] + rows_v[buf, jb + t, pl.ds(0, LANES)]
                accs[2 * t + 1] = (
                    accs[2 * t + 1] + rows_v[buf, jb + t, pl.ds(LANES, LANES)])
            return tuple(accs)

        accs = lax.fori_loop(0, L // U, j_body, (z,) * (2 * U))
        a0, a1 = accs[0], accs[1]
        for t in range(1, U):
            a0 = a0 + accs[2 * t]
            a1 = a1 + accs[2 * t + 1]
        lane = lax.rem(r, LANES)
        rvec = rcp_v[pl.ds(r - lane, LANES)]
        rb = jnp.take_along_axis(rvec, jnp.full((LANES,), lane), axis=0)
        p = lax.rem(r // SGR, 2)
        ro = lax.rem(r, SGR)
        out_v[p, pl.ds(ro * D, LANES)] = a0 * rb
        out_v[p, pl.ds(ro * D + LANES, LANES)] = a1 * rb

    h = ids_load_desc(0)
    h.start()
    h.wait()
    for t in range(NBUF - 1):
        issue(t, t)

    KPS = SGR // NBUF

    def k_body(k, carry):
        kin = lax.rem(k, KPS)
        s = k // KPS
        for t in range(NBUF):
            r = NBUF * k + t
            if t == 0:
                @pl.when((kin == 1) & (s + 1 < NSG))
                def _():
                    ids_load_desc(s + 1).start()

                @pl.when((kin == 0) & (s >= 2))
                def _():
                    out_write_op(s - 2, lambda h: h.wait())

                @pl.when((kin == KPS - 1) & (s + 1 < NSG))
                def _():
                    ids_load_desc(s + 1).wait()

            rr = r + NBUF - 1

            @pl.when(rr < BPW)
            def _():
                issue(rr, (t + NBUF - 1) % NBUF)

            wait(r, t)
            process(r, t)

            if t == NBUF - 1:
                @pl.when(kin == KPS - 1)
                def _():
                    out_write_op(s, lambda h: h.start())
        return carry

    lax.fori_loop(0, BPW // NBUF, k_body, 0)
    out_write_desc(NSG - 2, (NSG - 2) % 2).wait()
    out_write_desc(NSG - 1, (NSG - 1) % 2).wait()


def kernel(ids, lengths, table):
    ids_flat = ids.reshape(B * L)
    return _pool_kernel(ids_flat, lengths, table).reshape(B, D)
